# baseline (speedup 1.0000x reference)
import math

import jax
import jax.numpy as jnp
from jax import lax
from jax.experimental import pallas as pl
from jax.experimental.pallas import tpu as pltpu
from jax.experimental.pallas import tpu_sc as plsc

B, H, W = 2, 224, 224
GAUSS_DIM = 8
HEAD_IN = 64

_PITCH = 228
_NP = H * _PITCH
_PBLK = 57 * 128
_NPB = _NP // _PBLK

_SC_OWN = 14
_SC_HALO = 24


def _params_kernel(x_ref, w1_ref, b1_ref, w2_ref, b2_ref, out_ref):
    x = x_ref[0]
    w1 = w1_ref[...]
    w2 = w2_ref[...]
    feat = jax.lax.dot_general(w1, x, (((1,), (0,)), ((), ())),
                               preferred_element_type=jnp.float32)
    feat = jnp.maximum(feat + b1_ref[...], 0.0)
    pred = jax.lax.dot_general(w2, feat, (((1,), (0,)), ((), ())),
                               preferred_element_type=jnp.float32)
    pred = pred + b2_ref[...]

    rgb = pred[0:3]
    sg = 1.0 / (1.0 + jnp.exp(-pred[3:6]))
    theta = sg[0] * (2.0 * math.pi)
    sx = (sg[1] * 0.5 + 1e-6) * (W * 0.5)
    sy = (sg[2] * 0.5 + 1e-6) * (H * 0.5)
    off = jnp.tanh(pred[6:8])
    ct = jnp.cos(theta)
    st = jnp.sin(theta)
    sx2 = sx * sx
    sy2 = sy * sy
    ct2 = ct * ct
    st2 = st * st
    a = ct2 * sx2 + st2 * sy2
    bcov = ct * st * (sx2 - sy2)
    c = st2 * sx2 + ct2 * sy2
    det = a * c - bcov * bcov + 1e-12
    inv = 1.0 / det
    e0 = jnp.floor(off[0] - 0.5)
    e1 = jnp.floor(off[1] - 0.5)
    gx = e0 + 1.0 - off[0]
    gy = e1 + 1.0 - off[1]
    icx = -e0
    icy = -e1
    halfA = -0.5 * c * inv
    cb2 = bcov * inv
    halfC = -0.5 * a * inv
    out_ref[0] = jnp.concatenate(
        [gx[None], gy[None], icx[None], icy[None],
         halfA[None], cb2[None], halfC[None], rgb], axis=0)


def _sc_splat_body(pt_ref, out_ref, slab, fb):
    c = lax.axis_index("c")
    s = lax.axis_index("s")
    r0 = s * _SC_OWN
    lo = jnp.maximum(jnp.minimum(r0 - 4, H - _SC_HALO), 0)
    pltpu.sync_copy(pt_ref.at[c, :, pl.ds(lo, _SC_HALO), :], slab)
    offs = lo - r0

    zero16 = jnp.zeros((16,), jnp.float32)

    def zrow(r, _):
        for ch in range(3):
            for k in range(14):
                fb[ch, r, pl.ds(k * 16, 16)] = zero16
        return 0
    lax.fori_loop(0, _SC_OWN, zrow, 0)

    iotav = lax.iota(jnp.int32, 16)
    colv = iotav * 15

    def row_body(a, _):
        def grp_body(g, __):
            col = colv + g
            colm = col < W
            idxv = jnp.minimum(col, _PITCH - 1)
            gx = plsc.load_gather(slab.at[0, a], [idxv])
            gy = plsc.load_gather(slab.at[1, a], [idxv])
            icx = plsc.load_gather(slab.at[2, a], [idxv]).astype(jnp.int32)
            icy = plsc.load_gather(slab.at[3, a], [idxv]).astype(jnp.int32)
            hA = plsc.load_gather(slab.at[4, a], [idxv])
            cB2 = plsc.load_gather(slab.at[5, a], [idxv])
            hC = plsc.load_gather(slab.at[6, a], [idxv])
            rv = plsc.load_gather(slab.at[7, a], [idxv])
            gv = plsc.load_gather(slab.at[8, a], [idxv])
            bv = plsc.load_gather(slab.at[9, a], [idxv])
            cx = col - icx
            cyl = (a + offs) - icy
            uxs = [cx + dx for dx in range(-4, 5)]
            xms = [(ux >= 0) & (ux < W) & colm for ux in uxs]
            fxs = [gx + float(dx) for dx in range(-4, 5)]
            hfxs = [hA * fx for fx in fxs]

            for dy in range(-4, 5):
                uy = cyl + dy
                ym = (uy >= 0) & (uy < _SC_OWN)
                fy = gy + float(dy)
                xy = cB2 * fy
                ty = hC * fy * fy
                for dxi in range(9):
                    p = (hfxs[dxi] + xy) * fxs[dxi] + ty
                    alpha = jnp.minimum(jnp.exp(p), 1.0)
                    m = ym & xms[dxi]
                    ux = uxs[dxi]
                    plsc.addupdate_scatter(fb.at[0], [uy, ux], alpha * rv,
                                           mask=m)
                    plsc.addupdate_scatter(fb.at[1], [uy, ux], alpha * gv,
                                           mask=m)
                    plsc.addupdate_scatter(fb.at[2], [uy, ux], alpha * bv,
                                           mask=m)
            return 0
        lax.fori_loop(0, 15, grp_body, 0)
        return 0
    lax.fori_loop(0, _SC_HALO, row_body, 0)

    def crow(r, _):
        for ch in range(3):
            for k in range(14):
                v = fb[ch, r, pl.ds(k * 16, 16)]
                fb[ch, r, pl.ds(k * 16, 16)] = jnp.minimum(
                    jnp.maximum(v, 0.0), 1.0)
        return 0
    lax.fori_loop(0, _SC_OWN, crow, 0)
    pltpu.sync_copy(fb, out_ref.at[c, :, pl.ds(r0, _SC_OWN), :])


_sc_splat = pl.kernel(
    _sc_splat_body,
    out_type=jax.ShapeDtypeStruct((B, 3, H, W), jnp.float32),
    mesh=plsc.VectorSubcoreMesh(core_axis_name="c", subcore_axis_name="s"),
    scratch_types=[pltpu.VMEM((10, _SC_HALO, _PITCH), jnp.float32),
                   pltpu.VMEM((3, _SC_OWN, W), jnp.float32)],
    compiler_params=pltpu.CompilerParams(use_tc_tiling_on_sc=False,
                                         needs_layout_passes=False),
)


@jax.jit
def kernel(inp, enc_w, enc_b, head_w, head_b):
    xp = jnp.pad(inp, ((0, 0), (0, 0), (1, 2), (1, 3)))
    flat = xp.reshape(B, 3, 227 * _PITCH)
    slabs = [flat[:, :, dy * _PITCH + dx:dy * _PITCH + dx + _NP]
             for dy in range(3) for dx in range(3)]
    x = jnp.stack(slabs, axis=2).reshape(B, 27, _NP)
    x = jnp.pad(x, ((0, 0), (0, 5), (0, 0)))
    w1 = jnp.pad(enc_w.reshape(HEAD_IN, 27), ((0, 0), (0, 5)))
    w2 = head_w.reshape(GAUSS_DIM, HEAD_IN)
    b1 = enc_b.reshape(HEAD_IN, 1)
    b2 = head_b.reshape(GAUSS_DIM, 1)

    planes = pl.pallas_call(
        _params_kernel,
        grid=(B, _NPB),
        in_specs=[
            pl.BlockSpec((1, 32, _PBLK), lambda b, p: (b, 0, p)),
            pl.BlockSpec((HEAD_IN, 32), lambda b, p: (0, 0)),
            pl.BlockSpec((HEAD_IN, 1), lambda b, p: (0, 0)),
            pl.BlockSpec((GAUSS_DIM, HEAD_IN), lambda b, p: (0, 0)),
            pl.BlockSpec((GAUSS_DIM, 1), lambda b, p: (0, 0)),
        ],
        out_specs=pl.BlockSpec((1, 10, _PBLK), lambda b, p: (b, 0, p)),
        out_shape=jax.ShapeDtypeStruct((B, 10, _NP), jnp.float32),
        compiler_params=pltpu.CompilerParams(
            allow_input_fusion=[True, False, False, False, False]),
    )(x, w1, b1, w2, b2)

    return _sc_splat(planes.reshape(B, 10, H, _PITCH))


# speedup vs baseline: 1.4572x; 1.4572x over previous
import math

import jax
import jax.numpy as jnp
from jax import lax
from jax.experimental import pallas as pl
from jax.experimental.pallas import tpu as pltpu
from jax.experimental.pallas import tpu_sc as plsc

B, H, W = 2, 224, 224
GAUSS_DIM = 8
HEAD_IN = 64

HW = H * W
_PBLK = 6272
_NPB = HW // _PBLK

_SC_OWN = 14
_SC_HALO = 24


def _params_kernel(x_ref, w1_ref, b1_ref, w2_ref, b2_ref, out_ref):
    x = x_ref[0]
    w1 = w1_ref[...]
    w2 = w2_ref[...]
    feat = jax.lax.dot_general(w1, x, (((1,), (0,)), ((), ())),
                               preferred_element_type=jnp.float32)
    feat = jnp.maximum(feat + b1_ref[...], 0.0)
    pred = jax.lax.dot_general(w2, feat, (((1,), (0,)), ((), ())),
                               preferred_element_type=jnp.float32)
    pred = pred + b2_ref[...]

    rgb = pred[0:3]
    sg = 1.0 / (1.0 + jnp.exp(-pred[3:6]))
    theta = sg[0] * (2.0 * math.pi)
    sx = (sg[1] * 0.5 + 1e-6) * (W * 0.5)
    sy = (sg[2] * 0.5 + 1e-6) * (H * 0.5)
    off = jnp.tanh(pred[6:8])
    ct = jnp.cos(theta)
    st = jnp.sin(theta)
    sx2 = sx * sx
    sy2 = sy * sy
    ct2 = ct * ct
    st2 = st * st
    a = ct2 * sx2 + st2 * sy2
    bcov = ct * st * (sx2 - sy2)
    c = st2 * sx2 + ct2 * sy2
    det = a * c - bcov * bcov + 1e-12
    inv = 1.0 / det
    e0 = jnp.floor(off[0] - 0.5)
    e1 = jnp.floor(off[1] - 0.5)
    gx = e0 + 1.0 - off[0]
    gy = e1 + 1.0 - off[1]
    icx = -e0
    icy = -e1
    halfA = -0.5 * c * inv
    cb2 = bcov * inv
    halfC = -0.5 * a * inv
    out_ref[0] = jnp.concatenate(
        [gx[None], gy[None], icx[None], icy[None],
         halfA[None], cb2[None], halfC[None], rgb], axis=0)


def _sc_splat_body(pt_ref, out_ref, slab, fb):
    c = lax.axis_index("c")
    s = lax.axis_index("s")
    r0 = s * _SC_OWN
    lo = jnp.maximum(jnp.minimum(r0 - 4, H - _SC_HALO), 0)
    pltpu.sync_copy(pt_ref.at[c, :, pl.ds(lo, _SC_HALO), :], slab)
    offs = lo - r0

    zero16 = jnp.zeros((16,), jnp.float32)

    def zrow(r, _):
        for ch in range(3):
            for k in range(14):
                fb[ch, r, pl.ds(k * 16, 16)] = zero16
        return 0
    lax.fori_loop(0, _SC_OWN, zrow, 0)

    iotav = lax.iota(jnp.int32, 16)
    colv = iotav * 15

    def row_body(a, _):
        def grp_body(g, __):
            col = colv + g
            colm = col < W
            idxv = jnp.minimum(col, W - 1)
            gx = plsc.load_gather(slab.at[0, a], [idxv])
            gy = plsc.load_gather(slab.at[1, a], [idxv])
            icx = plsc.load_gather(slab.at[2, a], [idxv]).astype(jnp.int32)
            icy = plsc.load_gather(slab.at[3, a], [idxv]).astype(jnp.int32)
            hA = plsc.load_gather(slab.at[4, a], [idxv])
            cB2 = plsc.load_gather(slab.at[5, a], [idxv])
            hC = plsc.load_gather(slab.at[6, a], [idxv])
            rv = plsc.load_gather(slab.at[7, a], [idxv])
            gv = plsc.load_gather(slab.at[8, a], [idxv])
            bv = plsc.load_gather(slab.at[9, a], [idxv])
            cx = col - icx
            cyl = (a + offs) - icy
            uxs = [cx + dx for dx in range(-4, 5)]
            xms = [(ux >= 0) & (ux < W) & colm for ux in uxs]
            fxs = [gx + float(dx) for dx in range(-4, 5)]
            hfxs = [hA * fx for fx in fxs]

            for dy in range(-4, 5):
                uy = cyl + dy
                ym = (uy >= 0) & (uy < _SC_OWN)
                fy = gy + float(dy)
                xy = cB2 * fy
                ty = hC * fy * fy
                for dxi in range(9):
                    p = (hfxs[dxi] + xy) * fxs[dxi] + ty
                    alpha = jnp.minimum(jnp.exp(p), 1.0)
                    m = ym & xms[dxi]
                    ux = uxs[dxi]
                    plsc.addupdate_scatter(fb.at[0], [uy, ux], alpha * rv,
                                           mask=m)
                    plsc.addupdate_scatter(fb.at[1], [uy, ux], alpha * gv,
                                           mask=m)
                    plsc.addupdate_scatter(fb.at[2], [uy, ux], alpha * bv,
                                           mask=m)
            return 0
        lax.fori_loop(0, 15, grp_body, 0)
        return 0
    lax.fori_loop(0, _SC_HALO, row_body, 0)

    def crow(r, _):
        for ch in range(3):
            for k in range(14):
                v = fb[ch, r, pl.ds(k * 16, 16)]
                fb[ch, r, pl.ds(k * 16, 16)] = jnp.minimum(
                    jnp.maximum(v, 0.0), 1.0)
        return 0
    lax.fori_loop(0, _SC_OWN, crow, 0)
    pltpu.sync_copy(fb, out_ref.at[c, :, pl.ds(r0, _SC_OWN), :])


_sc_splat = pl.kernel(
    _sc_splat_body,
    out_type=jax.ShapeDtypeStruct((B, 3, H, W), jnp.float32),
    mesh=plsc.VectorSubcoreMesh(core_axis_name="c", subcore_axis_name="s"),
    scratch_types=[pltpu.VMEM((10, _SC_HALO, W), jnp.float32),
                   pltpu.VMEM((3, _SC_OWN, W), jnp.float32)],
    compiler_params=pltpu.CompilerParams(use_tc_tiling_on_sc=False,
                                         needs_layout_passes=False),
)


@jax.jit
def kernel(inp, enc_w, enc_b, head_w, head_b):
    xp = jnp.pad(inp, ((0, 0), (0, 0), (1, 1), (1, 1)))
    slabs = [xp[:, :, dy:dy + H, dx:dx + W]
             for dy in range(3) for dx in range(3)]
    x = jnp.stack(slabs, axis=2).reshape(B, 27, HW)
    x = jnp.pad(x, ((0, 0), (0, 5), (0, 0)))
    w1 = jnp.pad(enc_w.reshape(HEAD_IN, 27), ((0, 0), (0, 5)))
    w2 = head_w.reshape(GAUSS_DIM, HEAD_IN)
    b1 = enc_b.reshape(HEAD_IN, 1)
    b2 = head_b.reshape(GAUSS_DIM, 1)

    planes = pl.pallas_call(
        _params_kernel,
        grid=(B, _NPB),
        in_specs=[
            pl.BlockSpec((1, 32, _PBLK), lambda b, p: (b, 0, p)),
            pl.BlockSpec((HEAD_IN, 32), lambda b, p: (0, 0)),
            pl.BlockSpec((HEAD_IN, 1), lambda b, p: (0, 0)),
            pl.BlockSpec((GAUSS_DIM, HEAD_IN), lambda b, p: (0, 0)),
            pl.BlockSpec((GAUSS_DIM, 1), lambda b, p: (0, 0)),
        ],
        out_specs=pl.BlockSpec((1, 10, _PBLK), lambda b, p: (b, 0, p)),
        out_shape=jax.ShapeDtypeStruct((B, 10, HW), jnp.float32),
        compiler_params=pltpu.CompilerParams(
            allow_input_fusion=[True, False, False, False, False]),
    )(x, w1, b1, w2, b2)

    return _sc_splat(planes.reshape(B, 10, H, W))
